# baseline (speedup 1.0000x reference)
import functools

import jax
import jax.numpy as jnp
from jax import lax
from jax.experimental import pallas as pl
from jax.experimental.pallas import tpu as pltpu
from jax.experimental.pallas import tpu_sc as plsc

N = 1000000
L = 16
NW = 32
EW0 = 31248
EW31 = 31312
CHUNK = 256
KCH = 123

_DNUMS = lax.GatherDimensionNumbers(
    offset_dims=(), collapsed_slice_dims=(0,), start_index_map=(0,))


def _ple_kernel(x_hbm, coef_hbm, out_hbm, coef_v, out_v, x_v):
    wid = lax.axis_index("s") * 2 + lax.axis_index("c")
    base = wid * EW0
    ew = jnp.where(wid == NW - 1, EW31, EW0)

    pltpu.sync_copy(coef_hbm, coef_v)
    a = coef_v[pl.ds(0, L)]
    b = coef_v[pl.ds(L, L)]
    lo = coef_v[pl.ds(2 * L, L)]
    hi = coef_v[pl.ds(3 * L, L)]

    def chunk_body(k, _):
        off = jnp.minimum(k * CHUNK, ew - CHUNK)
        pltpu.sync_copy(x_hbm.at[pl.ds(base + off, CHUNK)], x_v)

        zero16 = jnp.zeros((L,), dtype=jnp.int32)

        def group_body(g, _):
            gbase = g * L
            for e in range(L):
                ridx = jnp.full((L,), gbase + e, dtype=jnp.int32)
                xe = plsc.load_gather(x_v, [ridx, zero16])
                enc = jnp.minimum(jnp.maximum(xe * a + b, lo), hi)
                out_v[gbase + e] = enc
            return 0

        lax.fori_loop(0, CHUNK // L, group_body, 0)
        pltpu.sync_copy(out_v, out_hbm.at[pl.ds(base + off, CHUNK)])
        return 0

    lax.fori_loop(0, KCH, chunk_body, 0)


@jax.jit
def _ple(x, coef):
    mesh = plsc.VectorSubcoreMesh(core_axis_name="c", subcore_axis_name="s")
    f = functools.partial(
        pl.kernel,
        mesh=mesh,
        out_type=jax.ShapeDtypeStruct((N, L), jnp.float32),
        compiler_params=pltpu.CompilerParams(needs_layout_passes=False),
        scratch_types=[
            pltpu.VMEM((4 * L,), jnp.float32),
            pltpu.VMEM((CHUNK, L), jnp.float32),
            pltpu.VMEM((CHUNK, 1), jnp.float32),
        ],
    )(_ple_kernel)
    return f(x, coef)


def kernel(x, bins):
    n_bins = bins.shape[0]
    lk = jnp.concatenate([bins, jnp.array([-1.0], dtype=bins.dtype)])
    prev = lk[:n_bins]
    nxt = lk[1 : n_bins + 1]
    a = 1.0 / (nxt - prev)
    b = -prev * a
    j = jnp.arange(n_bins)
    neg_inf = jnp.float32(-jnp.inf)
    pos_inf = jnp.float32(jnp.inf)
    lo = jnp.where(j == 0, neg_inf, jnp.zeros((n_bins,), jnp.float32))
    hi = jnp.ones((n_bins,), jnp.float32)
    last_pos = a[n_bins - 1] > 0
    lo = lo.at[n_bins - 1].set(jnp.where(last_pos, 0.0, neg_inf))
    hi = hi.at[n_bins - 1].set(jnp.where(last_pos, pos_inf, 0.0))
    coef = jnp.concatenate([a, b, lo, hi]).astype(jnp.float32)
    return _ple(x, coef)[:, None, :]


# speedup vs baseline: 9.5681x; 9.5681x over previous
import functools

import jax
import jax.numpy as jnp
from jax import lax
from jax.experimental import pallas as pl
from jax.experimental.pallas import tpu as pltpu
from jax.experimental.pallas import tpu_sc as plsc

N = 1000000
L = 16
NW = 32
EW0 = 31232
EW31 = 31808
CHUNK = 512
KCH0 = 61
KCH31 = 62
TAIL = 64
TAIL_OFF = N - TAIL


def _ple_kernel(x_hbm, coef_hbm, out_hbm, coef_v, x_v, out_v, x_vt, out_vt):
    wid = lax.axis_index("s") * 2 + lax.axis_index("c")
    base = wid * EW0
    last = wid == NW - 1
    nk = jnp.where(last, KCH31, KCH0)

    pltpu.sync_copy(coef_hbm, coef_v)

    def compute(xref, oref, ngroups):
        for j in range(L):
            aj = coef_v[pl.ds(0, 1), pl.ds(j * L, L)]
            bj = coef_v[pl.ds(0, 1), pl.ds((L + j) * L, L)]
            loj = coef_v[pl.ds(0, 1), pl.ds((2 * L + j) * L, L)]
            hij = coef_v[pl.ds(0, 1), pl.ds((3 * L + j) * L, L)]
            for g in range(ngroups):
                xv = xref[pl.ds(0, 1), pl.ds(g * L, L)]
                enc = jnp.minimum(jnp.maximum(xv * aj + bj, loj), hij)
                oref[pl.ds(j, 1), pl.ds(g * L, L)] = enc

    def chunk_body(k, _):
        off = base + k * CHUNK
        pltpu.sync_copy(x_hbm.at[pl.ds(0, 1), pl.ds(off, CHUNK)], x_v)
        compute(x_v, out_v, CHUNK // L)
        pltpu.sync_copy(out_v, out_hbm.at[pl.ds(0, L), pl.ds(off, CHUNK)])
        return 0

    lax.fori_loop(0, nk, chunk_body, 0)

    @pl.when(last)
    def _tail():
        pltpu.sync_copy(
            x_hbm.at[pl.ds(0, 1), pl.ds(TAIL_OFF, TAIL)], x_vt)
        compute(x_vt, out_vt, TAIL // L)
        pltpu.sync_copy(
            out_vt, out_hbm.at[pl.ds(0, L), pl.ds(TAIL_OFF, TAIL)])


@jax.jit
def _ple(xt, coef):
    mesh = plsc.VectorSubcoreMesh(core_axis_name="c", subcore_axis_name="s")
    f = functools.partial(
        pl.kernel,
        mesh=mesh,
        out_type=jax.ShapeDtypeStruct((L, N), jnp.float32),
        scratch_types=[
            pltpu.VMEM((1, 4 * L * L), jnp.float32),
            pltpu.VMEM((1, CHUNK), jnp.float32),
            pltpu.VMEM((L, CHUNK), jnp.float32),
            pltpu.VMEM((1, TAIL), jnp.float32),
            pltpu.VMEM((L, TAIL), jnp.float32),
        ],
    )(_ple_kernel)
    return f(xt, coef)


def kernel(x, bins):
    n_bins = bins.shape[0]
    lk = jnp.concatenate([bins, jnp.array([-1.0], dtype=bins.dtype)])
    prev = lk[:n_bins]
    nxt = lk[1 : n_bins + 1]
    a = 1.0 / (nxt - prev)
    b = -prev * a
    j = jnp.arange(n_bins)
    neg_inf = jnp.float32(-jnp.inf)
    pos_inf = jnp.float32(jnp.inf)
    lo = jnp.where(j == 0, neg_inf, jnp.zeros((n_bins,), jnp.float32))
    hi = jnp.ones((n_bins,), jnp.float32)
    last_pos = a[n_bins - 1] > 0
    lo = lo.at[n_bins - 1].set(jnp.where(last_pos, 0.0, neg_inf))
    hi = hi.at[n_bins - 1].set(jnp.where(last_pos, pos_inf, 0.0))
    coef = jnp.concatenate([a, b, lo, hi]).astype(jnp.float32)
    coef_splat = jnp.repeat(coef, n_bins).reshape(1, -1)
    out_t = _ple(x.reshape(1, N), coef_splat)
    return out_t.T[:, None, :]


# speedup vs baseline: 11.7421x; 1.2272x over previous
import functools

import jax
import jax.numpy as jnp
from jax import lax
from jax.experimental import pallas as pl
from jax.experimental.pallas import tpu as pltpu
from jax.experimental.pallas import tpu_sc as plsc

N = 1000000
L = 16
NW = 32
EW0 = 31232
EW31 = 31808
CHUNK = 512
KCH = 62
NPAIR = KCH // 2
TAIL = 64
TAIL_OFF = N - TAIL


def _ple_kernel(x_hbm, coef_hbm, out_hbm, coef_v,
                x_v0, x_v1, out_v0, out_v1, x_vt, out_vt,
                sx0, sx1, so0, so1):
    wid = lax.axis_index("s") * 2 + lax.axis_index("c")
    base = wid * EW0
    last = wid == NW - 1
    offcap = jnp.where(last, (KCH - 1) * CHUNK, (KCH - 2) * CHUNK)

    pltpu.sync_copy(coef_hbm, coef_v)

    def off(k):
        return base + jnp.minimum(k * CHUNK, offcap)

    def xsl(k):
        return x_hbm.at[pl.ds(0, 1), pl.ds(off(k), CHUNK)]

    def osl(k):
        return out_hbm.at[pl.ds(0, L), pl.ds(off(k), CHUNK)]

    def compute(xref, oref, ngroups):
        for j in range(L):
            aj = coef_v[pl.ds(0, 1), pl.ds(j * L, L)]
            bj = coef_v[pl.ds(0, 1), pl.ds((L + j) * L, L)]
            loj = coef_v[pl.ds(0, 1), pl.ds((2 * L + j) * L, L)]
            hij = coef_v[pl.ds(0, 1), pl.ds((3 * L + j) * L, L)]
            for g in range(ngroups):
                xv = xref[pl.ds(0, 1), pl.ds(g * L, L)]
                enc = jnp.minimum(jnp.maximum(xv * aj + bj, loj), hij)
                oref[pl.ds(j, 1), pl.ds(g * L, L)] = enc

    pltpu.async_copy(xsl(0), x_v0, sx0)

    def pair_body(i2, _):
        ka = 2 * i2
        kb = ka + 1
        pltpu.make_async_copy(xsl(ka), x_v0, sx0).wait()
        pltpu.async_copy(xsl(kb), x_v1, sx1)

        @pl.when(i2 > 0)
        def _():
            pltpu.make_async_copy(out_v0, osl(ka), so0).wait()
        compute(x_v0, out_v0, CHUNK // L)
        pltpu.async_copy(out_v0, osl(ka), so0)

        pltpu.make_async_copy(xsl(kb), x_v1, sx1).wait()

        @pl.when(i2 < NPAIR - 1)
        def _():
            pltpu.async_copy(xsl(ka + 2), x_v0, sx0)

        @pl.when(i2 > 0)
        def _():
            pltpu.make_async_copy(out_v1, osl(kb), so1).wait()
        compute(x_v1, out_v1, CHUNK // L)
        pltpu.async_copy(out_v1, osl(kb), so1)
        return 0

    lax.fori_loop(0, NPAIR, pair_body, 0)
    pltpu.make_async_copy(out_v0, osl(KCH - 2), so0).wait()
    pltpu.make_async_copy(out_v1, osl(KCH - 1), so1).wait()

    @pl.when(last)
    def _tail():
        pltpu.sync_copy(
            x_hbm.at[pl.ds(0, 1), pl.ds(TAIL_OFF, TAIL)], x_vt)
        compute(x_vt, out_vt, TAIL // L)
        pltpu.sync_copy(
            out_vt, out_hbm.at[pl.ds(0, L), pl.ds(TAIL_OFF, TAIL)])


@jax.jit
def _ple(xt, coef):
    mesh = plsc.VectorSubcoreMesh(core_axis_name="c", subcore_axis_name="s")
    f = functools.partial(
        pl.kernel,
        mesh=mesh,
        out_type=jax.ShapeDtypeStruct((L, N), jnp.float32),
        scratch_types=[
            pltpu.VMEM((1, 4 * L * L), jnp.float32),
            pltpu.VMEM((1, CHUNK), jnp.float32),
            pltpu.VMEM((1, CHUNK), jnp.float32),
            pltpu.VMEM((L, CHUNK), jnp.float32),
            pltpu.VMEM((L, CHUNK), jnp.float32),
            pltpu.VMEM((1, TAIL), jnp.float32),
            pltpu.VMEM((L, TAIL), jnp.float32),
            pltpu.SemaphoreType.DMA,
            pltpu.SemaphoreType.DMA,
            pltpu.SemaphoreType.DMA,
            pltpu.SemaphoreType.DMA,
        ],
    )(_ple_kernel)
    return f(xt, coef)


def kernel(x, bins):
    n_bins = bins.shape[0]
    lk = jnp.concatenate([bins, jnp.array([-1.0], dtype=bins.dtype)])
    prev = lk[:n_bins]
    nxt = lk[1 : n_bins + 1]
    a = 1.0 / (nxt - prev)
    b = -prev * a
    j = jnp.arange(n_bins)
    neg_inf = jnp.float32(-jnp.inf)
    pos_inf = jnp.float32(jnp.inf)
    lo = jnp.where(j == 0, neg_inf, jnp.zeros((n_bins,), jnp.float32))
    hi = jnp.ones((n_bins,), jnp.float32)
    last_pos = a[n_bins - 1] > 0
    lo = lo.at[n_bins - 1].set(jnp.where(last_pos, 0.0, neg_inf))
    hi = hi.at[n_bins - 1].set(jnp.where(last_pos, pos_inf, 0.0))
    coef = jnp.concatenate([a, b, lo, hi]).astype(jnp.float32)
    coef_splat = jnp.repeat(coef, n_bins).reshape(1, -1)
    out_t = _ple(x.reshape(1, N), coef_splat)
    return out_t.T[:, None, :]
